# Initial kernel scaffold; baseline (speedup 1.0000x reference)
import jax
import jax.numpy as jnp
from jax.experimental import pallas as pl


def kernel(pos, dirs, intensity, surf_z, surf_c, surf_n):
    raise NotImplementedError("write your pallas kernel here")


# speedup vs baseline: 1.5254x; 1.5254x over previous
import functools

import jax
import jax.numpy as jnp
from jax import lax
from jax.experimental import pallas as pl
from jax.experimental.pallas import tpu as pltpu
from jax.experimental.pallas import tpu_sc as plsc

NC = 2
NS = 16
NW = NC * NS
L = 16
C = 2048

_EPS = 1e-6
_HALF = 0.5
_THREEHALF = 1.5
_MAGIC = 0x5F3759DF


def _rsqrt(s):
    i = lax.bitcast_convert_type(s, jnp.int32)
    y = lax.bitcast_convert_type(_MAGIC - lax.shift_right_logical(i, 1),
                                 jnp.float32)
    for _ in range(3):
        y = y * (_THREEHALF - _HALF * s * y * y)
    return y


def _trace_chunk(v, pos_v, dirs_v, int_v, out_v, surf):
    iota = lax.iota(jnp.int32, L)
    i3 = iota * 3 + v * (3 * L)
    px = plsc.load_gather(pos_v, [i3])
    py = plsc.load_gather(pos_v, [i3 + 1])
    pz = plsc.load_gather(pos_v, [i3 + 2])
    dx = plsc.load_gather(dirs_v, [i3])
    dy = plsc.load_gather(dirs_v, [i3 + 1])
    dz = plsc.load_gather(dirs_v, [i3 + 2])
    inten = plsc.load_gather(int_v, [iota + v * L])

    rn = _rsqrt(dx * dx + dy * dy + dz * dz)
    dx, dy, dz = dx * rn, dy * rn, dz * rn

    for i in range(8):
        z, c, eta = surf[3 * i], surf[3 * i + 1], surf[3 * i + 2]
        eta2 = eta * eta
        adz = jnp.abs(dz)
        safe_dz = jnp.where(adz > _EPS, dz, 1.0)
        t = (z - pz) / safe_dz
        mask = (adz > _EPS) & (t > _EPS)
        t = jnp.where(mask, t, 0.0)
        nx_, ny_, nz_ = px + t * dx, py + t * dy, pz + t * dz
        gx, gy = -c * nx_, -c * ny_
        rn2 = _rsqrt(gx * gx + gy * gy + 1.0)
        gx, gy, gz = gx * rn2, gy * rn2, -rn2
        cos_i = jnp.clip(-(dx * gx + dy * gy + dz * gz), -1.0, 1.0)
        sin2 = eta2 * jnp.maximum(0.0, 1.0 - cos_i * cos_i)
        arg = jnp.maximum(1e-8, 1.0 - sin2)
        cos_t = arg * _rsqrt(arg)
        k = eta * cos_i - cos_t
        ndx = eta * dx + k * gx
        ndy = eta * dy + k * gy
        ndz = eta * dz + k * gz
        s3 = ndx * ndx + ndy * ndy + ndz * ndz
        nrm = jnp.maximum(s3 * _rsqrt(s3), 1e-8)
        ndx, ndy, ndz = ndx / nrm, ndy / nrm, ndz / nrm
        om = 1.0 - cos_i
        imod = jnp.clip(1.0 - 0.04 * om * om, 0.0, 1.0)
        nint = inten * imod
        px = jnp.where(mask, nx_, px)
        py = jnp.where(mask, ny_, py)
        pz = jnp.where(mask, nz_, pz)
        dx = jnp.where(mask, ndx, dx)
        dy = jnp.where(mask, ndy, dy)
        dz = jnp.where(mask, ndz, dz)
        inten = jnp.where(mask, nint, inten)

    i7 = iota * 7 + v * (7 * L)
    plsc.store_scatter(out_v, [i7], px)
    plsc.store_scatter(out_v, [i7 + 1], py)
    plsc.store_scatter(out_v, [i7 + 2], pz)
    plsc.store_scatter(out_v, [i7 + 3], dx)
    plsc.store_scatter(out_v, [i7 + 4], dy)
    plsc.store_scatter(out_v, [i7 + 5], dz)
    plsc.store_scatter(out_v, [i7 + 6], inten)


def _make_kernel(n):
    rw = n // NW
    chunks = rw // C
    assert rw * NW == n and chunks * C == rw

    mesh = plsc.VectorSubcoreMesh(core_axis_name="c", subcore_axis_name="s")

    @functools.partial(
        pl.kernel,
        out_type=jax.ShapeDtypeStruct((n * 7,), jnp.float32),
        mesh=mesh,
        compiler_params=pltpu.CompilerParams(needs_layout_passes=False),
        scratch_types=[
            pltpu.VMEM((3 * C,), jnp.float32),
            pltpu.VMEM((3 * C,), jnp.float32),
            pltpu.VMEM((C,), jnp.float32),
            pltpu.VMEM((7 * C,), jnp.float32),
            pltpu.VMEM((32,), jnp.float32),
        ],
    )
    def k(pos_hbm, dirs_hbm, int_hbm, surf_hbm, out_hbm,
          pos_v, dirs_v, int_v, out_v, surf_v):
        wid = lax.axis_index("s") * NC + lax.axis_index("c")
        base = wid * rw
        pltpu.sync_copy(surf_hbm, surf_v)
        sv0 = surf_v[pl.ds(0, L)]
        sv1 = surf_v[pl.ds(L, L)]
        surf = [sv0[j] if j < L else sv1[j - L] for j in range(24)]

        def chunk_body(g, carry):
            b = base + g * C
            pltpu.sync_copy(pos_hbm.at[pl.ds(b * 3, 3 * C)], pos_v)
            pltpu.sync_copy(dirs_hbm.at[pl.ds(b * 3, 3 * C)], dirs_v)
            pltpu.sync_copy(int_hbm.at[pl.ds(b, C)], int_v)

            def vec_body(v, cc):
                _trace_chunk(v, pos_v, dirs_v, int_v, out_v, surf)
                return cc

            lax.fori_loop(0, C // L, vec_body, 0)
            pltpu.sync_copy(out_v, out_hbm.at[pl.ds(b * 7, 7 * C)])
            return carry

        lax.fori_loop(0, chunks, chunk_body, 0)

    return k


def kernel(pos, dirs, intensity, surf_z, surf_c, surf_n):
    n = pos.shape[0]
    surf = jnp.concatenate(
        [jnp.stack([surf_z, surf_c, surf_n], axis=-1).reshape(-1),
         jnp.zeros((8,), jnp.float32)]).astype(jnp.float32)
    out = _make_kernel(n)(pos.reshape(-1), dirs.reshape(-1), intensity, surf)
    return out.reshape(n, 7)


# speedup vs baseline: 1.6830x; 1.1033x over previous
import functools

import jax
import jax.numpy as jnp
from jax import lax
from jax.experimental import pallas as pl
from jax.experimental.pallas import tpu as pltpu
from jax.experimental.pallas import tpu_sc as plsc

NC = 2
NS = 16
NW = NC * NS
L = 16
C = 2048
UNROLL = 2

_EPS = 1e-6
_MAGIC = 0x5F3759DF


def _rsqrt(s):
    i = lax.bitcast_convert_type(s, jnp.int32)
    y = lax.bitcast_convert_type(_MAGIC - lax.shift_right_logical(i, 1),
                                 jnp.float32)
    for _ in range(2):
        y = y * (1.5 - 0.5 * s * y * y)
    return y


def _trace_vec(v, pos_v, dirs_v, int_v, out_v, surf):
    iota = lax.iota(jnp.int32, L)
    i3 = iota * 3 + v * (3 * L)
    px = plsc.load_gather(pos_v, [i3])
    py = plsc.load_gather(pos_v, [i3 + 1])
    pz = plsc.load_gather(pos_v, [i3 + 2])
    dx = plsc.load_gather(dirs_v, [i3])
    dy = plsc.load_gather(dirs_v, [i3 + 1])
    dz = plsc.load_gather(dirs_v, [i3 + 2])
    inten = plsc.load_gather(int_v, [iota + v * L])

    rn = _rsqrt(dx * dx + dy * dy + dz * dz)
    dx, dy, dz = dx * rn, dy * rn, dz * rn

    for i in range(8):
        z, c, eta = surf[3 * i], surf[3 * i + 1], surf[3 * i + 2]
        eta2 = eta * eta
        adz = jnp.abs(dz)
        dzok = adz > _EPS
        safe_dz = jnp.where(dzok, dz, 1.0)
        t = (z - pz) / safe_dz
        mask = dzok & (t > _EPS)
        nx_, ny_, nz_ = px + t * dx, py + t * dy, pz + t * dz
        ux, uy = c * nx_, c * ny_
        rn2 = _rsqrt(ux * ux + uy * uy + 1.0)
        cos_i = jnp.clip((dx * ux + dy * uy + dz) * rn2, -1.0, 1.0)
        sin2 = eta2 * jnp.maximum(0.0, 1.0 - cos_i * cos_i)
        arg = jnp.maximum(1e-8, 1.0 - sin2)
        cos_t = arg * _rsqrt(arg)
        kk = (eta * cos_i - cos_t) * rn2
        ndx = eta * dx - kk * ux
        ndy = eta * dy - kk * uy
        ndz = eta * dz - kk
        rn3 = _rsqrt(ndx * ndx + ndy * ndy + ndz * ndz)
        ndx, ndy, ndz = ndx * rn3, ndy * rn3, ndz * rn3
        om = 1.0 - cos_i
        imod = jnp.clip(1.0 - 0.04 * om * om, 0.0, 1.0)
        nint = inten * imod
        px = jnp.where(mask, nx_, px)
        py = jnp.where(mask, ny_, py)
        pz = jnp.where(mask, nz_, pz)
        dx = jnp.where(mask, ndx, dx)
        dy = jnp.where(mask, ndy, dy)
        dz = jnp.where(mask, ndz, dz)
        inten = jnp.where(mask, nint, inten)

    i7 = iota * 7 + v * (7 * L)
    plsc.store_scatter(out_v, [i7], px)
    plsc.store_scatter(out_v, [i7 + 1], py)
    plsc.store_scatter(out_v, [i7 + 2], pz)
    plsc.store_scatter(out_v, [i7 + 3], dx)
    plsc.store_scatter(out_v, [i7 + 4], dy)
    plsc.store_scatter(out_v, [i7 + 5], dz)
    plsc.store_scatter(out_v, [i7 + 6], inten)


def _make_kernel(n):
    rw = n // NW
    chunks = rw // C
    assert rw * NW == n and chunks * C == rw

    mesh = plsc.VectorSubcoreMesh(core_axis_name="c", subcore_axis_name="s")

    @functools.partial(
        pl.kernel,
        out_type=jax.ShapeDtypeStruct((n * 7,), jnp.float32),
        mesh=mesh,
        compiler_params=pltpu.CompilerParams(needs_layout_passes=False),
        scratch_types=[
            pltpu.VMEM((3 * C,), jnp.float32),
            pltpu.VMEM((3 * C,), jnp.float32),
            pltpu.VMEM((C,), jnp.float32),
            pltpu.VMEM((7 * C,), jnp.float32),
            pltpu.VMEM((32,), jnp.float32),
        ],
    )
    def k(pos_hbm, dirs_hbm, int_hbm, surf_hbm, out_hbm,
          pos_v, dirs_v, int_v, out_v, surf_v):
        wid = lax.axis_index("s") * NC + lax.axis_index("c")
        base = wid * rw
        pltpu.sync_copy(surf_hbm, surf_v)
        sv0 = surf_v[pl.ds(0, L)]
        sv1 = surf_v[pl.ds(L, L)]
        surf = [sv0[j] if j < L else sv1[j - L] for j in range(24)]

        def chunk_body(g, carry):
            b = base + g * C
            pltpu.sync_copy(pos_hbm.at[pl.ds(b * 3, 3 * C)], pos_v)
            pltpu.sync_copy(dirs_hbm.at[pl.ds(b * 3, 3 * C)], dirs_v)
            pltpu.sync_copy(int_hbm.at[pl.ds(b, C)], int_v)

            def vec_body(w, cc):
                for j in range(UNROLL):
                    _trace_vec(w * UNROLL + j, pos_v, dirs_v, int_v, out_v,
                               surf)
                return cc

            lax.fori_loop(0, C // (L * UNROLL), vec_body, 0)
            pltpu.sync_copy(out_v, out_hbm.at[pl.ds(b * 7, 7 * C)])
            return carry

        lax.fori_loop(0, chunks, chunk_body, 0)

    return k


def kernel(pos, dirs, intensity, surf_z, surf_c, surf_n):
    n = pos.shape[0]
    surf = jnp.concatenate(
        [jnp.stack([surf_z, surf_c, surf_n], axis=-1).reshape(-1),
         jnp.zeros((8,), jnp.float32)]).astype(jnp.float32)
    out = _make_kernel(n)(pos.reshape(-1), dirs.reshape(-1), intensity, surf)
    return out.reshape(n, 7)


# speedup vs baseline: 3.7064x; 2.2022x over previous
import functools

import jax
import jax.numpy as jnp
from jax import lax
from jax.experimental import pallas as pl
from jax.experimental.pallas import tpu as pltpu
from jax.experimental.pallas import tpu_sc as plsc

NC = 2
NS = 16
NW = NC * NS
L = 16
C = 2048
UNROLL = 2

_EPS = 1e-6
_MAGIC = 0x5F3759DF


def _rsqrt(s):
    i = lax.bitcast_convert_type(s, jnp.int32)
    y = lax.bitcast_convert_type(_MAGIC - lax.shift_right_logical(i, 1),
                                 jnp.float32)
    for _ in range(2):
        y = y * (1.5 - 0.5 * s * y * y)
    return y


def _trace_vec(v, px_v, py_v, pz_v, dx_v, dy_v, dz_v, int_v, out_v, surf):
    iota = lax.iota(jnp.int32, L)
    sl = pl.ds(v * L, L)
    px = px_v[sl]
    py = py_v[sl]
    pz = pz_v[sl]
    dx = dx_v[sl]
    dy = dy_v[sl]
    dz = dz_v[sl]
    inten = int_v[sl]

    rn = _rsqrt(dx * dx + dy * dy + dz * dz)
    dx, dy, dz = dx * rn, dy * rn, dz * rn

    for i in range(8):
        z, c, eta = surf[3 * i], surf[3 * i + 1], surf[3 * i + 2]
        eta2 = eta * eta
        adz = jnp.abs(dz)
        dzok = adz > _EPS
        safe_dz = jnp.where(dzok, dz, 1.0)
        t = (z - pz) / safe_dz
        mask = dzok & (t > _EPS)
        nx_, ny_, nz_ = px + t * dx, py + t * dy, pz + t * dz
        ux, uy = c * nx_, c * ny_
        rn2 = _rsqrt(ux * ux + uy * uy + 1.0)
        cos_i = jnp.clip((dx * ux + dy * uy + dz) * rn2, -1.0, 1.0)
        sin2 = eta2 * jnp.maximum(0.0, 1.0 - cos_i * cos_i)
        arg = jnp.maximum(1e-8, 1.0 - sin2)
        cos_t = arg * _rsqrt(arg)
        kk = (eta * cos_i - cos_t) * rn2
        ndx = eta * dx - kk * ux
        ndy = eta * dy - kk * uy
        ndz = eta * dz - kk
        rn3 = _rsqrt(eta2 - (eta * cos_i) * (eta * cos_i) + arg)
        ndx, ndy, ndz = ndx * rn3, ndy * rn3, ndz * rn3
        om = 1.0 - cos_i
        imod = jnp.clip(1.0 - 0.04 * om * om, 0.0, 1.0)
        nint = inten * imod
        px = jnp.where(mask, nx_, px)
        py = jnp.where(mask, ny_, py)
        pz = jnp.where(mask, nz_, pz)
        dx = jnp.where(mask, ndx, dx)
        dy = jnp.where(mask, ndy, dy)
        dz = jnp.where(mask, ndz, dz)
        inten = jnp.where(mask, nint, inten)

    i7 = iota * 7 + v * (7 * L)
    plsc.store_scatter(out_v, [i7], px)
    plsc.store_scatter(out_v, [i7 + 1], py)
    plsc.store_scatter(out_v, [i7 + 2], pz)
    plsc.store_scatter(out_v, [i7 + 3], dx)
    plsc.store_scatter(out_v, [i7 + 4], dy)
    plsc.store_scatter(out_v, [i7 + 5], dz)
    plsc.store_scatter(out_v, [i7 + 6], inten)


def _make_kernel(n):
    rw = n // NW
    chunks = rw // C
    assert rw * NW == n and chunks * C == rw

    mesh = plsc.VectorSubcoreMesh(core_axis_name="c", subcore_axis_name="s")

    @functools.partial(
        pl.kernel,
        out_type=jax.ShapeDtypeStruct((n * 7,), jnp.float32),
        mesh=mesh,
        compiler_params=pltpu.CompilerParams(needs_layout_passes=False),
        scratch_types=[
            pltpu.VMEM((C,), jnp.float32),
            pltpu.VMEM((C,), jnp.float32),
            pltpu.VMEM((C,), jnp.float32),
            pltpu.VMEM((C,), jnp.float32),
            pltpu.VMEM((C,), jnp.float32),
            pltpu.VMEM((C,), jnp.float32),
            pltpu.VMEM((C,), jnp.float32),
            pltpu.VMEM((7 * C,), jnp.float32),
            pltpu.VMEM((32,), jnp.float32),
        ],
    )
    def k(posT_hbm, dirsT_hbm, int_hbm, surf_hbm, out_hbm,
          px_v, py_v, pz_v, dx_v, dy_v, dz_v, int_v, out_v, surf_v):
        wid = lax.axis_index("s") * NC + lax.axis_index("c")
        base = wid * rw
        pltpu.sync_copy(surf_hbm, surf_v)
        sv0 = surf_v[pl.ds(0, L)]
        sv1 = surf_v[pl.ds(L, L)]
        surf = [sv0[j] if j < L else sv1[j - L] for j in range(24)]

        def chunk_body(g, carry):
            b = base + g * C
            pltpu.sync_copy(posT_hbm.at[pl.ds(b, C)], px_v)
            pltpu.sync_copy(posT_hbm.at[pl.ds(n + b, C)], py_v)
            pltpu.sync_copy(posT_hbm.at[pl.ds(2 * n + b, C)], pz_v)
            pltpu.sync_copy(dirsT_hbm.at[pl.ds(b, C)], dx_v)
            pltpu.sync_copy(dirsT_hbm.at[pl.ds(n + b, C)], dy_v)
            pltpu.sync_copy(dirsT_hbm.at[pl.ds(2 * n + b, C)], dz_v)
            pltpu.sync_copy(int_hbm.at[pl.ds(b, C)], int_v)

            @plsc.parallel_loop(0, C // L, 1, unroll=UNROLL)
            def vec_body(v):
                _trace_vec(v, px_v, py_v, pz_v, dx_v, dy_v, dz_v, int_v,
                           out_v, surf)

            pltpu.sync_copy(out_v, out_hbm.at[pl.ds(b * 7, 7 * C)])
            return carry

        lax.fori_loop(0, chunks, chunk_body, 0)

    return k


def kernel(pos, dirs, intensity, surf_z, surf_c, surf_n):
    n = pos.shape[0]
    surf = jnp.concatenate(
        [jnp.stack([surf_z, surf_c, surf_n], axis=-1).reshape(-1),
         jnp.zeros((8,), jnp.float32)]).astype(jnp.float32)
    out = _make_kernel(n)(pos.T.reshape(-1), dirs.T.reshape(-1),
                          intensity, surf)
    return out.reshape(n, 7)
